# baseline (speedup 1.0000x reference)
import functools

import jax
import jax.numpy as jnp
import numpy as np
from jax import lax
from jax.experimental import pallas as pl
from jax.experimental.pallas import tpu as pltpu
from jax.experimental.pallas import tpu_sc as plsc

F = 26
V = 100000
D = 16
B = 4096
EPS = 1e-5
NPAIR = F * (F - 1) // 2
_ROWS_NP, _COLS_NP = np.triu_indices(F, k=1)

_NC = 2
_NS = 16
_NW = _NC * _NS
_BPW = B // _NW
_LPW = _BPW * F
_CHUNK = 256
_NCHUNK = _LPW // _CHUNK

_VPAD = 784 * 128


def _sc_gather_emb(e128, blk, r0a, cola):
    mesh = plsc.VectorSubcoreMesh(core_axis_name="c", subcore_axis_name="s")

    @functools.partial(
        pl.kernel,
        mesh=mesh,
        compiler_params=pltpu.CompilerParams(needs_layout_passes=False),
        out_type=jax.ShapeDtypeStruct((F * D, B), jnp.float32),
        scratch_types=[
            pltpu.VMEM((_CHUNK,), jnp.int32),
            pltpu.VMEM((_CHUNK,), jnp.int32),
            pltpu.VMEM((_CHUNK,), jnp.int32),
            pltpu.VMEM((_CHUNK, 128), jnp.float32),
            pltpu.VMEM((F * D, _BPW), jnp.float32),
            pltpu.SemaphoreType.DMA,
        ],
    )
    def gather_kernel(e128_hbm, blk_hbm, r0_hbm, col_hbm, et_out,
                      blk_v, r0_v, col_v, rows_v, out_v, sem):
        wid = lax.axis_index("s") * _NC + lax.axis_index("c")
        base = wid * _LPW
        lane_iota = lax.iota(jnp.int32, 16)

        def chunk_step(c, carry):
            cbase = base + c * _CHUNK
            pltpu.sync_copy(blk_hbm.at[pl.ds(cbase, _CHUNK)], blk_v)
            pltpu.sync_copy(r0_hbm.at[pl.ds(cbase, _CHUNK)], r0_v)
            pltpu.sync_copy(col_hbm.at[pl.ds(cbase, _CHUNK)], col_v)
            pltpu.async_copy(e128_hbm.at[blk_v], rows_v, sem).wait()

            def sub_step(s, carry2):
                s16 = s * 16
                row16 = s16 + lane_iota
                r016 = r0_v[pl.ds(s16, 16)]
                col16 = col_v[pl.ds(s16, 16)]
                for d in range(D):
                    dvec = lane_iota * 0 + d
                    vals = plsc.load_gather(rows_v, [row16, dvec])
                    plsc.store_scatter(out_v, [r016 + d, col16], vals)
                return carry2

            lax.fori_loop(0, _CHUNK // 16, sub_step, 0)
            return carry

        lax.fori_loop(0, _NCHUNK, chunk_step, 0)
        pltpu.sync_copy(out_v, et_out.at[:, pl.ds(wid * _BPW, _BPW)])

    return gather_kernel(e128, blk, r0a, cola)


def _sc_gather_lin(l128, xcols):
    mesh = plsc.VectorSubcoreMesh(core_axis_name="c", subcore_axis_name="s")

    @functools.partial(
        pl.kernel,
        mesh=mesh,
        compiler_params=pltpu.CompilerParams(needs_layout_passes=False),
        out_type=jax.ShapeDtypeStruct((F * B,), jnp.float32),
        scratch_types=[
            pltpu.VMEM((784, 128), jnp.float32),
            pltpu.VMEM((B,), jnp.int32),
            pltpu.VMEM((B,), jnp.float32),
        ],
    )
    def lin_kernel(l128_hbm, x_hbm, lt_out, slab_v, idx_v, val_v):
        wid = lax.axis_index("s") * _NC + lax.axis_index("c")

        @pl.when(wid < F)
        def _work():
            pltpu.sync_copy(l128_hbm.at[pl.ds(wid * 784, 784), :], slab_v)
            pltpu.sync_copy(x_hbm.at[wid], idx_v)

            def chunk(c, carry):
                b16 = c * 16
                v16 = idx_v[pl.ds(b16, 16)]
                vals = plsc.load_gather(
                    slab_v, [lax.shift_right_logical(v16, 7), v16 & 127])
                val_v[pl.ds(b16, 16)] = vals
                return carry

            lax.fori_loop(0, B // 16, chunk, 0)
            pltpu.sync_copy(val_v, lt_out.at[pl.ds(wid * B, B)])

    return lin_kernel(l128, xcols)


_BBLK = 128
_NBLK = B // _BBLK


def _dense_body(eT_ref, linT_ref, alpha_ref, beta_ref, out_ref, innerT_s):
    p = pl.program_id(0)
    lane0 = pl.multiple_of(p * _BBLK, _BBLK)

    svals = []
    for q in range(NPAIR):
        i = int(_ROWS_NP[q]) * D
        j = int(_COLS_NP[q]) * D
        prod = eT_ref[i:i + D, :] * eT_ref[j:j + D, :]
        svals.append(jnp.sum(prod, axis=0, keepdims=True))
    innerT_s[:, pl.ds(lane0, _BBLK)] = jnp.concatenate(svals, axis=0)

    @pl.when(p == _NBLK - 1)
    def _stats():
        inner = innerT_s[:, :]
        m = jnp.mean(inner, axis=1, keepdims=True)
        ex2 = jnp.mean(inner * inner, axis=1, keepdims=True)
        w = jnp.tanh(beta_ref[:, :]) * lax.rsqrt(ex2 - m * m + EPS)
        fm = jnp.sum(w * (inner - m), axis=0, keepdims=True)

        lin = linT_ref[:, :]
        lm = jnp.mean(lin, axis=1, keepdims=True)
        lex2 = jnp.mean(lin * lin, axis=1, keepdims=True)
        la = jnp.tanh(alpha_ref[:, :]) * lax.rsqrt(lex2 - lm * lm + EPS)
        lout = jnp.sum(la * (lin - lm), axis=0, keepdims=True)

        out_ref[:, :] = lax.transpose(jax.nn.sigmoid(lout + fm), (1, 0))


def _dense(eT, linT, alpha_col, beta_col):
    return pl.pallas_call(
        _dense_body,
        grid=(_NBLK,),
        out_shape=jax.ShapeDtypeStruct((B, 1), jnp.float32),
        in_specs=[
            pl.BlockSpec((F * D, _BBLK), lambda p: (0, p)),
            pl.BlockSpec((F, B), lambda p: (0, 0)),
            pl.BlockSpec((F, 1), lambda p: (0, 0)),
            pl.BlockSpec((NPAIR, 1), lambda p: (0, 0)),
        ],
        out_specs=pl.BlockSpec((B, 1), lambda p: (0, 0)),
        scratch_shapes=[
            pltpu.VMEM((NPAIR, B), jnp.float32),
        ],
    )(eT, linT, alpha_col, beta_col)


def kernel(x, lin_w, emb, alpha, beta):
    e128 = jnp.pad(emb, ((0, 0), (0, 0), (0, 128 - D))).reshape(F * V, 128)
    lpad = jnp.pad(lin_w.reshape(F, V), ((0, 0), (0, _VPAD - V)))
    l128 = lpad.reshape(F * 784, 128)

    xi = x.astype(jnp.int32)
    xflat = xi.reshape(-1)
    n = jnp.arange(B * F, dtype=jnp.int32)
    fld = n % F
    blk = fld * V + xflat
    eT = _sc_gather_emb(e128, blk, fld * D, (n // F) % _BPW)
    lt_flat = _sc_gather_lin(l128, xi.T)
    linT = lt_flat.reshape(F, B)
    return _dense(eT, linT, alpha.reshape(F, 1), beta.reshape(NPAIR, 1))


# speedup vs baseline: 1.2095x; 1.2095x over previous
import functools

import jax
import jax.numpy as jnp
import numpy as np
from jax import lax
from jax.experimental import pallas as pl
from jax.experimental.pallas import tpu as pltpu
from jax.experimental.pallas import tpu_sc as plsc

F = 26
V = 100000
D = 16
B = 4096
EPS = 1e-5
NPAIR = F * (F - 1) // 2
_ROWS_NP, _COLS_NP = np.triu_indices(F, k=1)

_NC = 2
_NS = 16
_NW = _NC * _NS
_BPW = B // _NW
_LPW = _BPW * F
_CHUNK = 256
_NCHUNK = _LPW // _CHUNK

_VPAD = 784 * 128


_ECHUNK = 16
_NECHUNK = _LPW // _ECHUNK
_VT = (V // 128) * 128


def _sc_gather_emb(etab, etail, r0a, c0a, lanea, taila, lanta):
    mesh = plsc.VectorSubcoreMesh(core_axis_name="c", subcore_axis_name="s")

    @functools.partial(
        pl.kernel,
        mesh=mesh,
        compiler_params=pltpu.CompilerParams(needs_layout_passes=False),
        out_type=jax.ShapeDtypeStruct((B * F * D,), jnp.float32),
        scratch_types=[
            pltpu.VMEM((_ECHUNK,), jnp.int32),
            pltpu.VMEM((_ECHUNK,), jnp.int32),
            pltpu.VMEM((_ECHUNK,), jnp.int32),
            pltpu.VMEM((_ECHUNK,), jnp.int32),
            pltpu.VMEM((_ECHUNK,), jnp.int32),
            pltpu.VMEM((F * D, V - _VT), jnp.float32),
            pltpu.VMEM((_ECHUNK * D, 128), jnp.float32),
            pltpu.VMEM((_ECHUNK * D,), jnp.float32),
            pltpu.SemaphoreType.DMA,
        ],
    )
    def gather_kernel(etab_hbm, etail_hbm, r0_hbm, c0_hbm, lane_hbm,
                      tail_hbm, lant_hbm, e_out,
                      r0_v, c0_v, lane_v, tail_v, lant_v,
                      tailbuf_v, slab_v, nat_v, sem):
        wid = lax.axis_index("s") * _NC + lax.axis_index("c")
        base = wid * _LPW
        lane_iota = lax.iota(jnp.int32, 16)
        pltpu.sync_copy(etail_hbm, tailbuf_v)

        def chunk_step(c, carry):
            cbase = base + c * _ECHUNK
            pltpu.sync_copy(r0_hbm.at[pl.ds(cbase, _ECHUNK)], r0_v)
            pltpu.sync_copy(c0_hbm.at[pl.ds(cbase, _ECHUNK)], c0_v)
            pltpu.sync_copy(lane_hbm.at[pl.ds(cbase, _ECHUNK)], lane_v)
            pltpu.sync_copy(tail_hbm.at[pl.ds(cbase, _ECHUNK)], tail_v)
            pltpu.sync_copy(lant_hbm.at[pl.ds(cbase, _ECHUNK)], lant_v)

            def fire(s, carry2):
                s16 = s * 16
                r016 = r0_v[pl.ds(s16, 16)]
                c016 = c0_v[pl.ds(s16, 16)]
                for k in range(16):
                    r0 = pl.multiple_of(r016[k], D)
                    c0 = pl.multiple_of(c016[k], 128)
                    pltpu.async_copy(
                        etab_hbm.at[pl.ds(r0, D), pl.ds(c0, 128)],
                        slab_v.at[pl.ds((s16 + k) * D, D), :], sem)
                return carry2

            lax.fori_loop(0, _ECHUNK // 16, fire, 0)

            def drain(s, carry2):
                pltpu.make_async_copy(
                    etab_hbm.at[pl.ds(0, D), pl.ds(0, 128)],
                    slab_v.at[pl.ds(s * D, D), :], sem).wait()
                return carry2

            lax.fori_loop(0, _ECHUNK, drain, 0)

            def select(s, carry2):
                s16 = s * 16
                lane16 = lane_v[pl.ds(s16, 16)]
                r016 = r0_v[pl.ds(s16, 16)]
                tail16 = tail_v[pl.ds(s16, 16)]
                lant16 = lant_v[pl.ds(s16, 16)]
                for k in range(16):
                    rows = (s16 + k) * D + lane_iota
                    lanes = lane_iota * 0 + lane16[k]
                    vmain = plsc.load_gather(slab_v, [rows, lanes])
                    vtail = plsc.load_gather(
                        tailbuf_v,
                        [r016[k] + lane_iota, lane_iota * 0 + lant16[k]])
                    nat_v[pl.ds((s16 + k) * D, D)] = jnp.where(
                        tail16[k] == 0, vmain, vtail)
                return carry2

            lax.fori_loop(0, _ECHUNK // 16, select, 0)
            pltpu.sync_copy(nat_v, e_out.at[pl.ds(cbase * D, _ECHUNK * D)])
            return carry

        lax.fori_loop(0, _NECHUNK, chunk_step, 0)

    return gather_kernel(etab, etail, r0a, c0a, lanea, taila, lanta)


def _sc_gather_lin(l128, xcols):
    mesh = plsc.VectorSubcoreMesh(core_axis_name="c", subcore_axis_name="s")

    @functools.partial(
        pl.kernel,
        mesh=mesh,
        compiler_params=pltpu.CompilerParams(needs_layout_passes=False),
        out_type=jax.ShapeDtypeStruct((F * B,), jnp.float32),
        scratch_types=[
            pltpu.VMEM((784, 128), jnp.float32),
            pltpu.VMEM((B,), jnp.int32),
            pltpu.VMEM((B,), jnp.float32),
        ],
    )
    def lin_kernel(l128_hbm, x_hbm, lt_out, slab_v, idx_v, val_v):
        wid = lax.axis_index("s") * _NC + lax.axis_index("c")

        @pl.when(wid < F)
        def _work():
            pltpu.sync_copy(l128_hbm.at[pl.ds(wid * 784, 784), :], slab_v)
            pltpu.sync_copy(x_hbm.at[wid], idx_v)

            def chunk(c, carry):
                b16 = c * 16
                v16 = idx_v[pl.ds(b16, 16)]
                vals = plsc.load_gather(
                    slab_v, [lax.shift_right_logical(v16, 7), v16 & 127])
                val_v[pl.ds(b16, 16)] = vals
                return carry

            lax.fori_loop(0, B // 16, chunk, 0)
            pltpu.sync_copy(val_v, lt_out.at[pl.ds(wid * B, B)])

    return lin_kernel(l128, xcols)


_BBLK = 128
_NBLK = B // _BBLK


def _dense_body(e2_ref, linT_ref, alpha_ref, beta_ref, out_ref, innerT_s):
    p = pl.program_id(0)
    lane0 = pl.multiple_of(p * _BBLK, _BBLK)

    eT_blk = lax.transpose(e2_ref[:, :], (1, 0))
    svals = []
    for q in range(NPAIR):
        i = int(_ROWS_NP[q]) * D
        j = int(_COLS_NP[q]) * D
        prod = eT_blk[i:i + D, :] * eT_blk[j:j + D, :]
        svals.append(jnp.sum(prod, axis=0, keepdims=True))
    innerT_s[:, pl.ds(lane0, _BBLK)] = jnp.concatenate(svals, axis=0)

    @pl.when(p == _NBLK - 1)
    def _stats():
        inner = innerT_s[:, :]
        m = jnp.mean(inner, axis=1, keepdims=True)
        ex2 = jnp.mean(inner * inner, axis=1, keepdims=True)
        w = jnp.tanh(beta_ref[:, :]) * lax.rsqrt(ex2 - m * m + EPS)
        fm = jnp.sum(w * (inner - m), axis=0, keepdims=True)

        lin = linT_ref[:, :]
        lm = jnp.mean(lin, axis=1, keepdims=True)
        lex2 = jnp.mean(lin * lin, axis=1, keepdims=True)
        la = jnp.tanh(alpha_ref[:, :]) * lax.rsqrt(lex2 - lm * lm + EPS)
        lout = jnp.sum(la * (lin - lm), axis=0, keepdims=True)

        out_ref[:, :] = lax.transpose(jax.nn.sigmoid(lout + fm), (1, 0))


def _dense(e2, linT, alpha_col, beta_col):
    return pl.pallas_call(
        _dense_body,
        grid=(_NBLK,),
        out_shape=jax.ShapeDtypeStruct((B, 1), jnp.float32),
        in_specs=[
            pl.BlockSpec((_BBLK, F * D), lambda p: (p, 0)),
            pl.BlockSpec((F, B), lambda p: (0, 0)),
            pl.BlockSpec((F, 1), lambda p: (0, 0)),
            pl.BlockSpec((NPAIR, 1), lambda p: (0, 0)),
        ],
        out_specs=pl.BlockSpec((B, 1), lambda p: (0, 0)),
        scratch_shapes=[
            pltpu.VMEM((NPAIR, B), jnp.float32),
        ],
    )(e2, linT, alpha_col, beta_col)


def kernel(x, lin_w, emb, alpha, beta):
    etab = emb.transpose(0, 2, 1).reshape(F * D, V)
    lpad = jnp.pad(lin_w.reshape(F, V), ((0, 0), (0, _VPAD - V)))
    l128 = lpad.reshape(F * 784, 128)

    xi = x.astype(jnp.int32)
    xflat = xi.reshape(-1)
    n = jnp.arange(B * F, dtype=jnp.int32)
    fld = n % F
    c0a = jnp.minimum(jax.lax.shift_right_logical(xflat, 7),
                      (_VT // 128) - 1) * 128
    e1d = _sc_gather_emb(
        etab, etab[:, _VT:], fld * D, c0a,
        jnp.minimum(xflat - c0a, 127),
        (xflat >= _VT).astype(jnp.int32),
        jnp.clip(xflat - _VT, 0, V - _VT - 1))
    e2 = e1d.reshape(B, F * D)
    lt_flat = _sc_gather_lin(l128, xi.T)
    linT = lt_flat.reshape(F, B)
    return _dense(e2, linT, alpha.reshape(F, 1), beta.reshape(NPAIR, 1))


# speedup vs baseline: 2.0588x; 1.7022x over previous
import functools

import jax
import jax.numpy as jnp
import numpy as np
from jax import lax
from jax.experimental import pallas as pl
from jax.experimental.pallas import tpu as pltpu
from jax.experimental.pallas import tpu_sc as plsc

F = 26
V = 100000
D = 16
B = 4096
EPS = 1e-5
NPAIR = F * (F - 1) // 2
_ROWS_NP, _COLS_NP = np.triu_indices(F, k=1)

_NC = 2
_NS = 16
_NW = _NC * _NS
_BPW = B // _NW
_LPW = _BPW * F
_CHUNK = 256
_NCHUNK = _LPW // _CHUNK

_VPAD = 784 * 128


_ECHUNK = 16
_NECHUNK = _LPW // _ECHUNK
_VT = (V // 128) * 128


def _sc_gather_emb(etab, etail, r0a, c0a, lanea, taila, lanta):
    mesh = plsc.VectorSubcoreMesh(core_axis_name="c", subcore_axis_name="s")

    @functools.partial(
        pl.kernel,
        mesh=mesh,
        compiler_params=pltpu.CompilerParams(needs_layout_passes=False),
        out_type=jax.ShapeDtypeStruct((B * F * D,), jnp.float32),
        scratch_types=[
            pltpu.VMEM((_LPW,), jnp.int32),
            pltpu.VMEM((_LPW,), jnp.int32),
            pltpu.VMEM((_LPW,), jnp.int32),
            pltpu.VMEM((_LPW,), jnp.int32),
            pltpu.VMEM((_LPW,), jnp.int32),
            pltpu.VMEM((F * D, V - _VT), jnp.float32),
            pltpu.VMEM((_ECHUNK * D, 128), jnp.float32),
            pltpu.VMEM((_LPW * D // 2,), jnp.float32),
            pltpu.SemaphoreType.DMA,
        ],
    )
    def gather_kernel(etab_hbm, etail_hbm, r0_hbm, c0_hbm, lane_hbm,
                      tail_hbm, lant_hbm, e_out,
                      r0_v, c0_v, lane_v, tail_v, lant_v,
                      tailbuf_v, slab_v, nat_v, sem):
        wid = lax.axis_index("s") * _NC + lax.axis_index("c")
        base = wid * _LPW
        lane_iota = lax.iota(jnp.int32, 16)
        pltpu.sync_copy(etail_hbm, tailbuf_v)
        pltpu.sync_copy(r0_hbm.at[pl.ds(base, _LPW)], r0_v)
        pltpu.sync_copy(c0_hbm.at[pl.ds(base, _LPW)], c0_v)
        pltpu.sync_copy(lane_hbm.at[pl.ds(base, _LPW)], lane_v)
        pltpu.sync_copy(tail_hbm.at[pl.ds(base, _LPW)], tail_v)
        pltpu.sync_copy(lant_hbm.at[pl.ds(base, _LPW)], lant_v)

        def chunk_step(c, carry):
            cb = c * _ECHUNK
            nb = (c % (_NECHUNK // 2)) * _ECHUNK

            def fire(s, carry2):
                s16 = cb + s * 16
                r016 = r0_v[pl.ds(s16, 16)]
                c016 = c0_v[pl.ds(s16, 16)]
                for k in range(16):
                    r0 = pl.multiple_of(r016[k], D)
                    c0 = pl.multiple_of(c016[k], 128)
                    pltpu.async_copy(
                        etab_hbm.at[pl.ds(r0, D), pl.ds(c0, 128)],
                        slab_v.at[pl.ds((s * 16 + k) * D, D), :], sem)
                return carry2

            lax.fori_loop(0, _ECHUNK // 16, fire, 0)
            pltpu.make_async_copy(
                etab_hbm.at[pl.ds(0, _ECHUNK * D), pl.ds(0, 128)],
                slab_v, sem).wait()

            def select(s, carry2):
                s16 = cb + s * 16
                lane16 = lane_v[pl.ds(s16, 16)]
                r016 = r0_v[pl.ds(s16, 16)]
                tail16 = tail_v[pl.ds(s16, 16)]
                lant16 = lant_v[pl.ds(s16, 16)]
                for k in range(16):
                    rows = (s * 16 + k) * D + lane_iota
                    lanes = lane_iota * 0 + lane16[k]
                    vmain = plsc.load_gather(slab_v, [rows, lanes])
                    vtail = plsc.load_gather(
                        tailbuf_v,
                        [r016[k] + lane_iota, lane_iota * 0 + lant16[k]])
                    nat_v[pl.ds((nb + s * 16 + k) * D, D)] = jnp.where(
                        tail16[k] == 0, vmain, vtail)
                return carry2

            lax.fori_loop(0, _ECHUNK // 16, select, 0)
            return carry

        half = _LPW * D // 2
        for h in range(2):
            lax.fori_loop(h * (_NECHUNK // 2), (h + 1) * (_NECHUNK // 2),
                          chunk_step, 0)
            pltpu.sync_copy(nat_v, e_out.at[pl.ds(base * D + h * half, half)])

    return gather_kernel(etab, etail, r0a, c0a, lanea, taila, lanta)


def _sc_gather_lin(l128, xcols):
    mesh = plsc.VectorSubcoreMesh(core_axis_name="c", subcore_axis_name="s")

    @functools.partial(
        pl.kernel,
        mesh=mesh,
        compiler_params=pltpu.CompilerParams(needs_layout_passes=False),
        out_type=jax.ShapeDtypeStruct((F * B,), jnp.float32),
        scratch_types=[
            pltpu.VMEM((784, 128), jnp.float32),
            pltpu.VMEM((B,), jnp.int32),
            pltpu.VMEM((B,), jnp.float32),
        ],
    )
    def lin_kernel(l128_hbm, x_hbm, lt_out, slab_v, idx_v, val_v):
        wid = lax.axis_index("s") * _NC + lax.axis_index("c")

        @pl.when(wid < F)
        def _work():
            pltpu.sync_copy(l128_hbm.at[pl.ds(wid * 784, 784), :], slab_v)
            pltpu.sync_copy(x_hbm.at[wid], idx_v)

            def chunk(c, carry):
                b16 = c * 16
                v16 = idx_v[pl.ds(b16, 16)]
                vals = plsc.load_gather(
                    slab_v, [lax.shift_right_logical(v16, 7), v16 & 127])
                val_v[pl.ds(b16, 16)] = vals
                return carry

            lax.fori_loop(0, B // 16, chunk, 0)
            pltpu.sync_copy(val_v, lt_out.at[pl.ds(wid * B, B)])

    return lin_kernel(l128, xcols)


_BBLK = 128
_NBLK = B // _BBLK


def _dense_body(e2_ref, linT_ref, alpha_ref, beta_ref, out_ref, innerT_s):
    p = pl.program_id(0)
    lane0 = pl.multiple_of(p * _BBLK, _BBLK)

    eT_blk = lax.transpose(e2_ref[:, :], (1, 0))
    svals = []
    for q in range(NPAIR):
        i = int(_ROWS_NP[q]) * D
        j = int(_COLS_NP[q]) * D
        prod = eT_blk[i:i + D, :] * eT_blk[j:j + D, :]
        svals.append(jnp.sum(prod, axis=0, keepdims=True))
    innerT_s[:, pl.ds(lane0, _BBLK)] = jnp.concatenate(svals, axis=0)

    @pl.when(p == _NBLK - 1)
    def _stats():
        inner = innerT_s[:, :]
        m = jnp.mean(inner, axis=1, keepdims=True)
        ex2 = jnp.mean(inner * inner, axis=1, keepdims=True)
        w = jnp.tanh(beta_ref[:, :]) * lax.rsqrt(ex2 - m * m + EPS)
        fm = jnp.sum(w * (inner - m), axis=0, keepdims=True)

        lin = linT_ref[:, :]
        lm = jnp.mean(lin, axis=1, keepdims=True)
        lex2 = jnp.mean(lin * lin, axis=1, keepdims=True)
        la = jnp.tanh(alpha_ref[:, :]) * lax.rsqrt(lex2 - lm * lm + EPS)
        lout = jnp.sum(la * (lin - lm), axis=0, keepdims=True)

        out_ref[:, :] = lax.transpose(jax.nn.sigmoid(lout + fm), (1, 0))


def _dense(e2, linT, alpha_col, beta_col):
    return pl.pallas_call(
        _dense_body,
        grid=(_NBLK,),
        out_shape=jax.ShapeDtypeStruct((B, 1), jnp.float32),
        in_specs=[
            pl.BlockSpec((_BBLK, F * D), lambda p: (p, 0)),
            pl.BlockSpec((F, B), lambda p: (0, 0)),
            pl.BlockSpec((F, 1), lambda p: (0, 0)),
            pl.BlockSpec((NPAIR, 1), lambda p: (0, 0)),
        ],
        out_specs=pl.BlockSpec((B, 1), lambda p: (0, 0)),
        scratch_shapes=[
            pltpu.VMEM((NPAIR, B), jnp.float32),
        ],
    )(e2, linT, alpha_col, beta_col)


def kernel(x, lin_w, emb, alpha, beta):
    etab = emb.transpose(0, 2, 1).reshape(F * D, V)
    lpad = jnp.pad(lin_w.reshape(F, V), ((0, 0), (0, _VPAD - V)))
    l128 = lpad.reshape(F * 784, 128)

    xi = x.astype(jnp.int32)
    xflat = xi.reshape(-1)
    n = jnp.arange(B * F, dtype=jnp.int32)
    fld = n % F
    c0a = jnp.minimum(jax.lax.shift_right_logical(xflat, 7),
                      (_VT // 128) - 1) * 128
    e1d = _sc_gather_emb(
        etab, etab[:, _VT:], fld * D, c0a,
        jnp.minimum(xflat - c0a, 127),
        (xflat >= _VT).astype(jnp.int32),
        jnp.clip(xflat - _VT, 0, V - _VT - 1))
    e2 = e1d.reshape(B, F * D)
    lt_flat = _sc_gather_lin(l128, xi.T)
    linT = lt_flat.reshape(F, B)
    return _dense(e2, linT, alpha.reshape(F, 1), beta.reshape(NPAIR, 1))
